# Initial kernel scaffold; baseline (speedup 1.0000x reference)
import jax
import jax.numpy as jnp
from jax.experimental import pallas as pl


def kernel(pos_v, pos_u, neg_u, v_weight, u_weight):
    raise NotImplementedError("write your pallas kernel here")


# speedup vs baseline: 3.2637x; 3.2637x over previous
import functools

import jax
import jax.numpy as jnp
from jax import lax
from jax.experimental import pallas as pl
from jax.experimental.pallas import tpu as pltpu
from jax.experimental.pallas import tpu_sc as plsc

V = 100000
D = 128
B = 16384
K = 5

_info = plsc.get_sparse_core_info()
NC = _info.num_cores
NS = _info.num_subcores
L = _info.num_lanes
NW = NC * NS
BPW = B // NW
CHUNK = 64
NCHUNK = BPW // CHUNK
GROUPS = CHUNK // L
DC = D // L
NT = 1 + K


def _neg_log_sigmoid(x):
    m = jnp.minimum(x, 0.0)
    z = jnp.exp(-jnp.abs(x))
    s = z / (z + 2.0)
    s2 = s * s
    log1p_z = 2.0 * s * (1.0 + s2 * (1.0 / 3.0 + s2 * (1.0 / 5.0 + s2 * (1.0 / 7.0 + s2 * (1.0 / 9.0)))))
    return log1p_z - m


def _sc_body(pos_v_h, pos_u_h, neg_h, vw_h, uw_h, out_h,
             vidx, uidx, nidx, vrows, urows, nrows, lossbuf, sem):
    cid = lax.axis_index("c")
    sid = lax.axis_index("s")
    wid = sid * NC + cid

    pltpu.sync_copy(pos_v_h.at[pl.ds(wid * BPW, BPW)], vidx)
    pltpu.sync_copy(pos_u_h.at[pl.ds(wid * BPW, BPW)], uidx)
    pltpu.sync_copy(neg_h.at[pl.ds(wid * (BPW * K), BPW * K)], nidx)

    iota = lax.iota(jnp.int32, L)

    def group_body(g, loss):
        base = g * L
        zero = jnp.zeros((L,), jnp.float32)
        scores = [zero] * NT
        for ii in range(L):
            b = base + ii
            acc_p = None
            acc_n = [None] * K
            for c in range(DC):
                vv = vrows[b, pl.ds(c * L, L)]
                uu = urows[b, pl.ds(c * L, L)]
                p = vv * uu
                acc_p = p if acc_p is None else acc_p + p
                for k in range(K):
                    q = nrows[k, b, pl.ds(c * L, L)] * vv
                    acc_n[k] = q if acc_n[k] is None else acc_n[k] + q
            lane = iota == ii
            scores[0] = jnp.where(lane, jnp.sum(acc_p), scores[0])
            for k in range(K):
                scores[1 + k] = jnp.where(lane, jnp.sum(acc_n[k]), scores[1 + k])
        for t in range(NT):
            x = scores[t] if t == 0 else -scores[t]
            loss = loss + _neg_log_sigmoid(x)
        return loss

    def chunk_body(j, loss):
        cps = [
            pltpu.async_copy(vw_h.at[vidx.at[pl.ds(j * CHUNK, CHUNK)]], vrows, sem),
            pltpu.async_copy(uw_h.at[uidx.at[pl.ds(j * CHUNK, CHUNK)]], urows, sem),
        ]
        for k in range(K):
            cps.append(pltpu.async_copy(
                uw_h.at[nidx.at[pl.ds((j * K + k) * CHUNK, CHUNK)]], nrows.at[k], sem))
        for cp in cps:
            cp.wait()
        return lax.fori_loop(0, GROUPS, group_body, loss)

    loss = lax.fori_loop(0, NCHUNK, chunk_body, jnp.zeros((L,), jnp.float32))
    lossbuf[...] = loss
    pltpu.sync_copy(lossbuf, out_h.at[wid])


_sc_call = functools.partial(
    pl.kernel,
    out_type=jax.ShapeDtypeStruct((NW, L), jnp.float32),
    mesh=plsc.VectorSubcoreMesh(core_axis_name="c", subcore_axis_name="s"),
    compiler_params=pltpu.CompilerParams(needs_layout_passes=False),
    scratch_types=[
        pltpu.VMEM((BPW,), jnp.int32),
        pltpu.VMEM((BPW,), jnp.int32),
        pltpu.VMEM((BPW * K,), jnp.int32),
        pltpu.VMEM((CHUNK, D), jnp.float32),
        pltpu.VMEM((CHUNK, D), jnp.float32),
        pltpu.VMEM((K, CHUNK, D), jnp.float32),
        pltpu.VMEM((L,), jnp.float32),
        pltpu.SemaphoreType.DMA,
    ],
)(_sc_body)


def kernel(pos_v, pos_u, neg_u, v_weight, u_weight):
    pos_v = pos_v.astype(jnp.int32)
    pos_u = pos_u.astype(jnp.int32)
    neg = (neg_u.astype(jnp.int32)
           .reshape(NW, NCHUNK, CHUNK, K)
           .transpose(0, 1, 3, 2)
           .reshape(NW * BPW * K))
    partials = _sc_call(pos_v, pos_u, neg, v_weight, u_weight)
    return jnp.sum(partials)
